# baseline (speedup 1.0000x reference)
import functools

import jax
import jax.numpy as jnp
from jax import lax
from jax.experimental import pallas as pl
from jax.experimental.pallas import tpu as pltpu
from jax.experimental.pallas import tpu_sc as plsc

_NC = 2
_NS = 16
_NW = _NC * _NS
_SB = 64
_L = 16


def _sc_aggregate(x, a_rows, a_cols, a_val, n_pad):
    d = x.shape[1]
    e_pad = a_val.shape[0]
    ept = e_pad // _NW
    nfull = ept // _SB
    ntail = (ept % _SB) // _L
    ngrp = d // _L
    rpt = (-(-n_pad // _NS) + 7) // 8 * 8
    mesh = plsc.VectorSubcoreMesh(core_axis_name="c", subcore_axis_name="s")

    @functools.partial(
        pl.kernel,
        mesh=mesh,
        out_type=jax.ShapeDtypeStruct((_NC, n_pad, d), jnp.float32),
        scratch_types=[
            pltpu.VMEM((ept,), jnp.int32),
            pltpu.VMEM((ept,), jnp.int32),
            pltpu.VMEM((ept,), jnp.float32),
            pltpu.VMEM((2 * _SB, d), jnp.float32),
            pltpu.VMEM((_SB,), jnp.int32),
            pltpu.VMEM((_SB,), jnp.int32),
            pltpu.VMEM((_SB,), jnp.int32),
            pltpu.VMEM((_SB,), jnp.int32),
            pltpu.VMEM((_L,), jnp.int32),
            pltpu.VMEM((_L,), jnp.int32),
            pltpu.VMEM_SHARED((n_pad, d), jnp.float32),
            pltpu.SemaphoreType.DMA,
            pltpu.SemaphoreType.DMA,
            pltpu.SemaphoreType.DMA,
            pltpu.SemaphoreType.DMA,
        ],
    )
    def k(x_hbm, arows_hbm, acols_hbm, aval_hbm, out_hbm,
          cols_v, rows_v, vals_v, gbuf, cidx0, cidx1, ridx0, ridx1, ctail,
          rtail, acc, sem0, sem1, sem2, sem3):
        cidx = (cidx0, cidx1)
        ridx = (ridx0, ridx1)
        gsem = (sem0, sem1)
        ssem = (sem2, sem3)
        c = lax.axis_index("c")
        s = lax.axis_index("s")
        w = c * _NS + s

        zero_row = jnp.zeros((_L,), jnp.float32)

        def zero_body(i, carry):
            for j in range(ngrp):
                gbuf[i, pl.ds(j * _L, _L)] = zero_row
            return carry

        lax.fori_loop(0, 2 * _SB, zero_body, 0)
        base = jnp.minimum(s * rpt, n_pad - rpt)
        for blk in range(rpt // (2 * _SB)):
            pltpu.sync_copy(gbuf, acc.at[pl.ds(base + blk * 2 * _SB, 2 * _SB)])
        rem = rpt % (2 * _SB)
        if rem:
            pltpu.sync_copy(gbuf.at[pl.ds(0, rem)],
                            acc.at[pl.ds(base + (rpt // (2 * _SB)) * 2 * _SB,
                                         rem)])

        eb = w * ept
        pltpu.sync_copy(acols_hbm.at[pl.ds(eb, ept)], cols_v)
        pltpu.sync_copy(arows_hbm.at[pl.ds(eb, ept)], rows_v)
        pltpu.sync_copy(aval_hbm.at[pl.ds(eb, ept)], vals_v)
        plsc.subcore_barrier()

        def gather_issue(sb, half):
            for g in range(_SB // _L):
                cidx[half][pl.ds(g * _L, _L)] = \
                    cols_v[pl.ds(sb * _SB + g * _L, _L)]
            pltpu.async_copy(x_hbm.at[cidx[half]],
                             gbuf.at[pl.ds(half * _SB, _SB)], gsem[half])

        def gather_wait(half):
            pltpu.make_async_copy(x_hbm.at[cidx[half]],
                                  gbuf.at[pl.ds(half * _SB, _SB)],
                                  gsem[half]).wait()

        def scale_stage(sb, half):
            def scale_body(g, carry2):
                off = sb * _SB + g * _L
                ridx[half][pl.ds(g * _L, _L)] = rows_v[pl.ds(off, _L)]
                vv = vals_v[pl.ds(off, _L)]
                for l in range(_L):
                    v = vv[l]
                    i = half * _SB + g * _L + l
                    for j in range(ngrp):
                        sl = pl.ds(j * _L, _L)
                        gbuf[i, sl] = gbuf[i, sl] * v
                return carry2

            lax.fori_loop(0, _SB // _L, scale_body, 0)

        def scatter_start(half):
            pltpu.async_copy(gbuf.at[pl.ds(half * _SB, _SB)],
                            acc.at[ridx[half]], ssem[half], add=True)

        def scatter_wait(half):
            pltpu.make_async_copy(gbuf.at[pl.ds(half * _SB, _SB)],
                                  acc.at[ridx[half]], ssem[half]).wait()

        if nfull:
            gather_issue(0, 0)

            def batch_pair(p, carry):
                for h in range(2):
                    sb = p * 2 + h

                    @pl.when(sb == 0)
                    def _():
                        if nfull > 1:
                            gather_issue(1, 1)

                    @pl.when(jnp.logical_and(sb >= 1, sb + 1 < nfull))
                    def _():
                        scatter_wait(1 - h)
                        gather_issue(sb + 1, 1 - h)

                    gather_wait(h)
                    scale_stage(sb, h)
                    scatter_start(h)
                return carry

            lax.fori_loop(0, nfull // 2, batch_pair, 0)
            if nfull % 2:
                sb = nfull - 1
                gather_wait(sb % 2)
                scale_stage(sb, sb % 2)
                scatter_start(sb % 2)
            scatter_wait((nfull - 1) % 2)
            if nfull > 1:
                scatter_wait(nfull % 2)

        for t in range(ntail):
            off = nfull * _SB + t * _L
            ctail[pl.ds(0, _L)] = cols_v[pl.ds(off, _L)]
            pltpu.async_copy(x_hbm.at[ctail], gbuf.at[pl.ds(0, _L)], gsem[0])
            pltpu.make_async_copy(x_hbm.at[ctail], gbuf.at[pl.ds(0, _L)],
                                  gsem[0]).wait()
            rtail[pl.ds(0, _L)] = rows_v[pl.ds(off, _L)]
            vv = vals_v[pl.ds(off, _L)]
            for l in range(_L):
                v = vv[l]
                for j in range(ngrp):
                    sl = pl.ds(j * _L, _L)
                    gbuf[l, sl] = gbuf[l, sl] * v
            pltpu.sync_copy(gbuf.at[pl.ds(0, _L)], acc.at[rtail], add=True)

        plsc.subcore_barrier()
        pltpu.sync_copy(acc.at[pl.ds(base, rpt)], out_hbm.at[c, pl.ds(base, rpt)])

    return k(x, a_rows, a_cols, a_val)


def _tc_transform(partial, w_mat, bias_row):
    _, m, d = partial.shape
    bm = 1000 if m % 1000 == 0 else 8

    def body(p0_ref, p1_ref, w_ref, b_ref, o_ref):
        agg = p0_ref[0] + p1_ref[0]
        h = lax.dot_general(agg, w_ref[...], (((1,), (1,)), ((), ())),
                            preferred_element_type=jnp.float32)
        o_ref[...] = jnp.maximum(h + b_ref[...], 0.0)

    return pl.pallas_call(
        body,
        grid=(m // bm,),
        in_specs=[
            pl.BlockSpec((1, bm, d), lambda i: (0, i, 0)),
            pl.BlockSpec((1, bm, d), lambda i: (1, i, 0)),
            pl.BlockSpec((d, d), lambda i: (0, 0)),
            pl.BlockSpec((1, d), lambda i: (0, 0)),
        ],
        out_specs=pl.BlockSpec((bm, d), lambda i: (i, 0)),
        out_shape=jax.ShapeDtypeStruct((m, d), jnp.float32),
    )(partial, partial, w_mat, bias_row)


def kernel(x, A_indices, A_values, A_shape, W, bias):
    n, d = x.shape
    e = A_values.shape[0]

    group = _L * _NW
    e_pad = ((e + group - 1) // group) * group
    pad = e_pad - e
    a_rows = jnp.pad(A_indices[0], (0, pad))
    a_cols = jnp.pad(A_indices[1], (0, pad))
    a_val = jnp.pad(A_values, (0, pad)) if pad else A_values

    n_pad = ((n + 7) // 8) * 8
    partial = _sc_aggregate(x, a_rows, a_cols, a_val, n_pad)

    residual = (jnp.asarray(A_shape) - n).astype(jnp.float32)
    bias_row = (bias + residual).reshape(1, d)
    out_full = _tc_transform(partial, W, bias_row)
    return out_full[:n]


# speedup vs baseline: 1.1191x; 1.1191x over previous
import functools

import jax
import jax.numpy as jnp
from jax import lax
from jax.experimental import pallas as pl
from jax.experimental.pallas import tpu as pltpu
from jax.experimental.pallas import tpu_sc as plsc

_NC = 2
_NS = 16
_NW = _NC * _NS
_SB = 48
_NB = 3
_L = 16


def _sc_aggregate(x, a_rows, a_cols, a_val, n_pad):
    d = x.shape[1]
    e_pad = a_val.shape[0]
    ept = e_pad // _NW
    nfull = ept // _SB
    ntail = (ept % _SB) // _L
    ngrp = d // _L
    rpt = (-(-n_pad // _NS) + 7) // 8 * 8
    mesh = plsc.VectorSubcoreMesh(core_axis_name="c", subcore_axis_name="s")

    @functools.partial(
        pl.kernel,
        mesh=mesh,
        out_type=jax.ShapeDtypeStruct((_NC, n_pad, d), jnp.float32),
        scratch_types=[
            pltpu.VMEM((ept,), jnp.int32),
            pltpu.VMEM((ept,), jnp.int32),
            pltpu.VMEM((ept,), jnp.float32),
            pltpu.VMEM((_NB * _SB, d), jnp.float32),
        ] + [pltpu.VMEM((_SB,), jnp.int32) for _ in range(2 * _NB)]
          + [
            pltpu.VMEM((_L,), jnp.int32),
            pltpu.VMEM((_L,), jnp.int32),
            pltpu.VMEM_SHARED((n_pad, d), jnp.float32),
        ] + [pltpu.SemaphoreType.DMA] * (2 * _NB),
    )
    def k(x_hbm, arows_hbm, acols_hbm, aval_hbm, out_hbm,
          cols_v, rows_v, vals_v, gbuf, *rest):
        cidx = rest[:_NB]
        ridx = rest[_NB:2 * _NB]
        ctail, rtail, acc = rest[2 * _NB:2 * _NB + 3]
        gsem = rest[2 * _NB + 3:2 * _NB + 3 + _NB]
        ssem = rest[2 * _NB + 3 + _NB:]
        c = lax.axis_index("c")
        s = lax.axis_index("s")
        w = c * _NS + s

        zero_row = jnp.zeros((_L,), jnp.float32)

        def zero_body(i, carry):
            for j in range(ngrp):
                gbuf[i, pl.ds(j * _L, _L)] = zero_row
            return carry

        nzr = _NB * _SB
        lax.fori_loop(0, nzr, zero_body, 0)
        base = jnp.minimum(s * rpt, n_pad - rpt)
        for blk in range(rpt // nzr):
            pltpu.sync_copy(gbuf, acc.at[pl.ds(base + blk * nzr, nzr)])
        rem = rpt % nzr
        if rem:
            pltpu.sync_copy(gbuf.at[pl.ds(0, rem)],
                            acc.at[pl.ds(base + (rpt // nzr) * nzr, rem)])

        eb = w * ept
        pltpu.sync_copy(acols_hbm.at[pl.ds(eb, ept)], cols_v)
        pltpu.sync_copy(arows_hbm.at[pl.ds(eb, ept)], rows_v)
        pltpu.sync_copy(aval_hbm.at[pl.ds(eb, ept)], vals_v)
        plsc.subcore_barrier()

        def gather_issue(sb, k_):
            for g in range(_SB // _L):
                cidx[k_][pl.ds(g * _L, _L)] = \
                    cols_v[pl.ds(sb * _SB + g * _L, _L)]
            pltpu.async_copy(x_hbm.at[cidx[k_]],
                             gbuf.at[pl.ds(k_ * _SB, _SB)], gsem[k_])

        def gather_wait(k_):
            pltpu.make_async_copy(x_hbm.at[cidx[k_]],
                                  gbuf.at[pl.ds(k_ * _SB, _SB)],
                                  gsem[k_]).wait()

        def scale_stage(sb, k_):
            def scale_body(g, carry2):
                off = sb * _SB + g * _L
                ridx[k_][pl.ds(g * _L, _L)] = rows_v[pl.ds(off, _L)]
                vv = vals_v[pl.ds(off, _L)]
                for l in range(_L):
                    v = vv[l]
                    i = k_ * _SB + g * _L + l
                    for j in range(ngrp):
                        sl = pl.ds(j * _L, _L)
                        gbuf[i, sl] = gbuf[i, sl] * v
                return carry2

            lax.fori_loop(0, _SB // _L, scale_body, 0)

        def scatter_start(k_):
            pltpu.async_copy(gbuf.at[pl.ds(k_ * _SB, _SB)],
                            acc.at[ridx[k_]], ssem[k_], add=True)

        def scatter_wait(k_):
            pltpu.make_async_copy(gbuf.at[pl.ds(k_ * _SB, _SB)],
                                  acc.at[ridx[k_]], ssem[k_]).wait()

        if nfull:
            def slot(sb, k_, first, live):
                k2 = (k_ + 2) % _NB
                if first:
                    if nfull > 2:
                        gather_issue(2, 2)
                elif live:
                    scatter_wait(k2)
                    gather_issue(sb + 2, k2)
                gather_wait(k_)
                scale_stage(sb, k_)
                scatter_start(k_)

            for k_ in range(min(2, nfull)):
                gather_issue(k_, k_)

            nloop = (nfull - 1) // _NB

            def batch_trip(p, carry):
                for h in range(_NB):
                    sb = p * _NB + h

                    @pl.when(sb == 0)
                    def _():
                        slot(sb, h, True, False)

                    @pl.when(jnp.logical_and(sb >= 1, sb + 2 < nfull))
                    def _():
                        slot(sb, h, False, True)

                    @pl.when(jnp.logical_and(sb >= 1, sb + 2 >= nfull))
                    def _():
                        slot(sb, h, False, False)
                return carry

            lax.fori_loop(0, nloop, batch_trip, 0)
            for sb in range(nloop * _NB, nfull):
                slot(sb, sb % _NB, sb == 0, sb + 2 < nfull)
            for m in range(max(nfull - _NB, 0), nfull):
                scatter_wait(m % _NB)

        for t in range(ntail):
            off = nfull * _SB + t * _L
            ctail[pl.ds(0, _L)] = cols_v[pl.ds(off, _L)]
            pltpu.async_copy(x_hbm.at[ctail], gbuf.at[pl.ds(0, _L)], gsem[0])
            pltpu.make_async_copy(x_hbm.at[ctail], gbuf.at[pl.ds(0, _L)],
                                  gsem[0]).wait()
            rtail[pl.ds(0, _L)] = rows_v[pl.ds(off, _L)]
            vv = vals_v[pl.ds(off, _L)]
            for l in range(_L):
                v = vv[l]
                for j in range(ngrp):
                    sl = pl.ds(j * _L, _L)
                    gbuf[l, sl] = gbuf[l, sl] * v
            pltpu.sync_copy(gbuf.at[pl.ds(0, _L)], acc.at[rtail], add=True)

        plsc.subcore_barrier()
        pltpu.sync_copy(acc.at[pl.ds(base, rpt)], out_hbm.at[c, pl.ds(base, rpt)])

    return k(x, a_rows, a_cols, a_val)


def _tc_transform(partial, w_mat, bias_row):
    _, m, d = partial.shape
    bm = 1000 if m % 1000 == 0 else 8

    def body(p0_ref, p1_ref, w_ref, b_ref, o_ref):
        agg = p0_ref[0] + p1_ref[0]
        h = lax.dot_general(agg, w_ref[...], (((1,), (1,)), ((), ())),
                            preferred_element_type=jnp.float32)
        o_ref[...] = jnp.maximum(h + b_ref[...], 0.0)

    return pl.pallas_call(
        body,
        grid=(m // bm,),
        in_specs=[
            pl.BlockSpec((1, bm, d), lambda i: (0, i, 0)),
            pl.BlockSpec((1, bm, d), lambda i: (1, i, 0)),
            pl.BlockSpec((d, d), lambda i: (0, 0)),
            pl.BlockSpec((1, d), lambda i: (0, 0)),
        ],
        out_specs=pl.BlockSpec((bm, d), lambda i: (i, 0)),
        out_shape=jax.ShapeDtypeStruct((m, d), jnp.float32),
    )(partial, partial, w_mat, bias_row)


def kernel(x, A_indices, A_values, A_shape, W, bias):
    n, d = x.shape
    e = A_values.shape[0]

    group = _L * _NW
    e_pad = ((e + group - 1) // group) * group
    pad = e_pad - e
    a_rows = jnp.pad(A_indices[0], (0, pad))
    a_cols = jnp.pad(A_indices[1], (0, pad))
    a_val = jnp.pad(A_values, (0, pad)) if pad else A_values

    n_pad = ((n + 7) // 8) * 8
    partial = _sc_aggregate(x, a_rows, a_cols, a_val, n_pad)

    residual = (jnp.asarray(A_shape) - n).astype(jnp.float32)
    bias_row = (bias + residual).reshape(1, d)
    out_full = _tc_transform(partial, W, bias_row)
    return out_full[:n]
